# Initial kernel scaffold; baseline (speedup 1.0000x reference)
import jax
import jax.numpy as jnp
from jax.experimental import pallas as pl


def kernel(h, edge_index, W1, b1, W2, b2, W3, b3):
    raise NotImplementedError("write your pallas kernel here")


# speedup vs baseline: 15.6687x; 15.6687x over previous
import functools

import jax
import jax.numpy as jnp
from jax import lax
from jax.experimental import pallas as pl
from jax.experimental.pallas import tpu as pltpu
from jax.experimental.pallas import tpu_sc as plsc

_N = 10000
_E = 320000
_D = 128
_H = 16
_K = 8

_NC = 2
_NS = 16
_NW = _NC * _NS
_EW = _E // _NW
_C = 2000
_NCH = _EW // _C
_SUB = 500
_NSUB = _C // _SUB
_G = _C // 16


def _tables_body(h_ref, w1_ref, b1_ref, w2_ref, b2_ref, a_ref, b_ref):
    h1 = lax.dot_general(h_ref[...], w1_ref[...],
                         (((1,), (1,)), ((), ())),
                         preferred_element_type=jnp.float32) + b1_ref[...]
    w2 = w2_ref[...]
    a_ref[...] = lax.dot_general(h1, w2[:, :_H],
                                 (((1,), (1,)), ((), ())),
                                 preferred_element_type=jnp.float32) + b2_ref[...]
    b_ref[...] = lax.dot_general(h1, w2[:, _H:],
                                 (((1,), (1,)), ((), ())),
                                 preferred_element_type=jnp.float32)


_tables_call = pl.pallas_call(
    _tables_body,
    out_shape=[
        jax.ShapeDtypeStruct((_N, _K), jnp.float32),
        jax.ShapeDtypeStruct((_N, _K), jnp.float32),
    ],
)


def _edge_body(a_hbm, b_hbm, src_hbm, dst_hbm, w3_hbm, out_hbm,
               idx_s, idx_d, arows, brows, sco, w3v, sem):
    c = lax.axis_index("c")
    s = lax.axis_index("s")
    wid = s * _NC + c
    base = wid * _EW

    pltpu.sync_copy(w3_hbm, w3v)
    w3bc = [w3v[k] for k in range(_K)]

    def chunk(ch, carry):
        e0 = pl.multiple_of(base + ch * _C, 256)
        r0 = pl.multiple_of(e0 // _SUB, 4)
        pltpu.sync_copy(src_hbm.at[pl.ds(r0, _NSUB)], idx_s)
        pltpu.sync_copy(dst_hbm.at[pl.ds(r0, _NSUB)], idx_d)
        copies = []
        for j in range(_NSUB):
            copies.append(pltpu.async_copy(
                a_hbm.at[idx_s.at[j]], arows.at[pl.ds(j * _SUB, _SUB)], sem))
            copies.append(pltpu.async_copy(
                b_hbm.at[idx_d.at[j]], brows.at[pl.ds(j * _SUB, _SUB)], sem))
        for cp in copies:
            cp.wait()

        def group(g, gcarry):
            rows = g * 16 + lax.iota(jnp.int32, 16)
            acc = jnp.zeros((16,), jnp.float32)
            for k in range(_K):
                kk = jnp.full((16,), k, jnp.int32)
                av = plsc.load_gather(arows, [rows, kk])
                bv = plsc.load_gather(brows, [rows, kk])
                acc = acc + jnp.maximum(av + bv, 0.0) * w3bc[k]
            sco[pl.ds(g * 16, 16)] = acc
            return gcarry

        lax.fori_loop(0, _G, group, 0)
        pltpu.sync_copy(sco, out_hbm.at[pl.ds(e0, _C)])
        return carry

    lax.fori_loop(0, _NCH, chunk, 0)


_edge_call = functools.partial(
    pl.kernel,
    out_type=jax.ShapeDtypeStruct((_E,), jnp.float32),
    mesh=plsc.VectorSubcoreMesh(core_axis_name="c", subcore_axis_name="s",
                                num_cores=_NC, num_subcores=_NS),
    compiler_params=pltpu.CompilerParams(
        needs_layout_passes=False, use_tc_tiling_on_sc=False),
    scratch_types=[
        pltpu.VMEM((_NSUB, _SUB), jnp.int32),
        pltpu.VMEM((_NSUB, _SUB), jnp.int32),
        pltpu.VMEM((_C, _K), jnp.float32),
        pltpu.VMEM((_C, _K), jnp.float32),
        pltpu.VMEM((_C,), jnp.float32),
        pltpu.VMEM((_K, 16), jnp.float32),
        pltpu.SemaphoreType.DMA,
    ],
)(_edge_body)


def _norm_body(s_ref, o_ref):
    sv = s_ref[...]
    mn = jnp.min(sv)
    mx = jnp.max(sv)
    o_ref[...] = (sv - mn) / (mx - mn)


_norm_call = pl.pallas_call(
    _norm_body,
    out_shape=jax.ShapeDtypeStruct((_E // 128, 128), jnp.float32),
)


def kernel(h, edge_index, W1, b1, W2, b2, W3, b3):
    a_tab, b_tab = _tables_call(h, W1, b1.reshape(1, _H), W2, b2.reshape(1, _K))
    src2d = edge_index[0].reshape(_E // _SUB, _SUB)
    dst2d = edge_index[1].reshape(_E // _SUB, _SUB)
    w3bc = jnp.broadcast_to(W3.reshape(_K, 1), (_K, 16)) + jnp.zeros((_K, 16))
    scores = _edge_call(a_tab, b_tab, src2d, dst2d, w3bc)
    out2d = _norm_call(scores.reshape(_E // 128, 128))
    return out2d.reshape(_E, 1)


# speedup vs baseline: 15.8424x; 1.0111x over previous
import functools

import jax
import jax.numpy as jnp
from jax import lax
from jax.experimental import pallas as pl
from jax.experimental.pallas import tpu as pltpu
from jax.experimental.pallas import tpu_sc as plsc

_N = 10000
_E = 320000
_D = 128
_H = 16
_K = 8

_NC = 2
_NS = 16
_NW = _NC * _NS
_EW = _E // _NW
_C = 2000
_NCH = _EW // _C
_SUB = 400
_NSUB = _C // _SUB
_G = _C // 16

_NB = 10
_RB = _N // _NB


def _tables_body(h_ref, w1_ref, b1_ref, w2_ref, b2_ref, w3_ref,
                 a_ref, b_ref, w3b_ref):
    h1 = lax.dot_general(h_ref[...], w1_ref[...],
                         (((1,), (1,)), ((), ())),
                         preferred_element_type=jnp.float32) + b1_ref[...][None, :]
    w2 = w2_ref[...]
    a_ref[...] = lax.dot_general(h1, w2[:, :_H],
                                 (((1,), (1,)), ((), ())),
                                 preferred_element_type=jnp.float32) + b2_ref[...][None, :]
    b_ref[...] = lax.dot_general(h1, w2[:, _H:],
                                 (((1,), (1,)), ((), ())),
                                 preferred_element_type=jnp.float32)
    w3b_ref[...] = lax.dot_general(w3_ref[...], jnp.ones((1, 16), jnp.float32),
                                   (((0,), (0,)), ((), ())),
                                   preferred_element_type=jnp.float32)


_tables_call = pl.pallas_call(
    _tables_body,
    grid=(_NB,),
    in_specs=[
        pl.BlockSpec((_RB, _D), lambda i: (i, 0)),
        pl.BlockSpec((_H, _D), lambda i: (0, 0)),
        pl.BlockSpec((_H,), lambda i: (0,)),
        pl.BlockSpec((_K, 2 * _H), lambda i: (0, 0)),
        pl.BlockSpec((_K,), lambda i: (0,)),
        pl.BlockSpec((1, _K), lambda i: (0, 0)),
    ],
    out_specs=[
        pl.BlockSpec((_RB, _K), lambda i: (i, 0)),
        pl.BlockSpec((_RB, _K), lambda i: (i, 0)),
        pl.BlockSpec((_K, 16), lambda i: (0, 0)),
    ],
    out_shape=[
        jax.ShapeDtypeStruct((_N, _K), jnp.float32),
        jax.ShapeDtypeStruct((_N, _K), jnp.float32),
        jax.ShapeDtypeStruct((_K, 16), jnp.float32),
    ],
)


def _edge_body(a_hbm, b_hbm, src_hbm, dst_hbm, w3_hbm, out_hbm,
               idx_s, idx_d, arows, brows, sco, w3v, sem):
    c = lax.axis_index("c")
    s = lax.axis_index("s")
    wid = s * _NC + c
    base = wid * _EW

    pltpu.sync_copy(w3_hbm, w3v)
    w3bc = [w3v[k] for k in range(_K)]

    def chunk(ch, carry):
        e0 = pl.multiple_of(base + ch * _C, 256)
        pltpu.sync_copy(src_hbm.at[pl.ds(e0, _C)], idx_s)
        pltpu.sync_copy(dst_hbm.at[pl.ds(e0, _C)], idx_d)
        copies = []
        for j in range(_NSUB):
            copies.append(pltpu.async_copy(
                a_hbm.at[idx_s.at[pl.ds(j * _SUB, _SUB)]],
                arows.at[pl.ds(j * _SUB, _SUB)], sem))
            copies.append(pltpu.async_copy(
                b_hbm.at[idx_d.at[pl.ds(j * _SUB, _SUB)]],
                brows.at[pl.ds(j * _SUB, _SUB)], sem))
        for cp in copies:
            cp.wait()

        def group(g, gcarry):
            rows = g * 16 + lax.iota(jnp.int32, 16)
            acc = jnp.zeros((16,), jnp.float32)
            for k in range(_K):
                kk = jnp.full((16,), k, jnp.int32)
                av = plsc.load_gather(arows, [rows, kk])
                bv = plsc.load_gather(brows, [rows, kk])
                acc = acc + jnp.maximum(av + bv, 0.0) * w3bc[k]
            sco[pl.ds(g * 16, 16)] = acc
            return gcarry

        lax.fori_loop(0, _G, group, 0)
        pltpu.sync_copy(sco, out_hbm.at[pl.ds(e0, _C)])
        return carry

    lax.fori_loop(0, _NCH, chunk, 0)


_edge_call = functools.partial(
    pl.kernel,
    out_type=jax.ShapeDtypeStruct((_E,), jnp.float32),
    mesh=plsc.VectorSubcoreMesh(core_axis_name="c", subcore_axis_name="s",
                                num_cores=_NC, num_subcores=_NS),
    compiler_params=pltpu.CompilerParams(
        needs_layout_passes=False, use_tc_tiling_on_sc=False),
    scratch_types=[
        pltpu.VMEM((_C,), jnp.int32),
        pltpu.VMEM((_C,), jnp.int32),
        pltpu.VMEM((_C, _K), jnp.float32),
        pltpu.VMEM((_C, _K), jnp.float32),
        pltpu.VMEM((_C,), jnp.float32),
        pltpu.VMEM((_K, 16), jnp.float32),
        pltpu.SemaphoreType.DMA,
    ],
)(_edge_body)


def _norm_body(s_ref, o_ref):
    sv = s_ref[...]
    mn = jnp.min(sv)
    mx = jnp.max(sv)
    o_ref[...] = (sv - mn) / (mx - mn)


_norm_call = pl.pallas_call(
    _norm_body,
    out_shape=jax.ShapeDtypeStruct((_E // 128, 128), jnp.float32),
)


def kernel(h, edge_index, W1, b1, W2, b2, W3, b3):
    a_tab, b_tab, w3b = _tables_call(h, W1, b1, W2, b2, W3)
    src = edge_index[0]
    dst = edge_index[1]
    scores = _edge_call(a_tab, b_tab, src, dst, w3b)
    out2d = _norm_call(scores.reshape(_E // 128, 128))
    return out2d.reshape(_E, 1)


# speedup vs baseline: 19.3428x; 1.2209x over previous
import functools

import jax
import jax.numpy as jnp
from jax import lax
from jax.experimental import pallas as pl
from jax.experimental.pallas import tpu as pltpu
from jax.experimental.pallas import tpu_sc as plsc

_N = 10000
_E = 320000
_D = 128
_H = 16
_K = 8

_NC = 2
_NS = 16
_NW = _NC * _NS
_EW = _E // _NW
_C = 2000
_NCH = _EW // _C
_SUB = 400
_NSUB = _C // _SUB
_G = _C // 16


def _tables_body(h_ref, w1_ref, b1_ref, w2_ref, b2_ref, w3_ref,
                 a_ref, b_ref, w3b_ref):
    h1 = lax.dot_general(h_ref[...], w1_ref[...],
                         (((1,), (1,)), ((), ())),
                         preferred_element_type=jnp.float32) + b1_ref[...][None, :]
    w2 = w2_ref[...]
    a_ref[...] = lax.dot_general(h1, w2[:, :_H],
                                 (((1,), (1,)), ((), ())),
                                 preferred_element_type=jnp.float32) + b2_ref[...][None, :]
    b_ref[...] = lax.dot_general(h1, w2[:, _H:],
                                 (((1,), (1,)), ((), ())),
                                 preferred_element_type=jnp.float32)
    w3b_ref[...] = lax.dot_general(w3_ref[...], jnp.ones((1, 16), jnp.float32),
                                   (((0,), (0,)), ((), ())),
                                   preferred_element_type=jnp.float32)


_tables_call = pl.pallas_call(
    _tables_body,
    out_shape=[
        jax.ShapeDtypeStruct((_N, _K), jnp.float32),
        jax.ShapeDtypeStruct((_N, _K), jnp.float32),
        jax.ShapeDtypeStruct((_K, 16), jnp.float32),
    ],
)


def _edge_body(a_hbm, b_hbm, src_hbm, dst_hbm, w3_hbm, out_hbm,
               idx_s, idx_d, arows0, brows0, arows1, brows1,
               sco0, sco1, w3v, gsem, wsem):
    c = lax.axis_index("c")
    s = lax.axis_index("s")
    wid = s * _NC + c
    base = pl.multiple_of(wid * _EW, 256)

    pltpu.sync_copy(w3_hbm, w3v)
    iot = lax.iota(jnp.int32, 16)
    kd = [(iot + cc) & (_K - 1) for cc in range(_K)]
    w3d = [plsc.load_gather(w3v, [kd[cc], iot]) for cc in range(_K)]

    pltpu.sync_copy(src_hbm.at[pl.ds(base, _EW)], idx_s)
    pltpu.sync_copy(dst_hbm.at[pl.ds(base, _EW)], idx_d)

    bufs = [(arows0, brows0, sco0), (arows1, brows1, sco1)]

    def issue(ch):
        ar, br, _ = bufs[ch % 2]
        cps = []
        for j in range(_NSUB):
            off = ch * _C + j * _SUB
            cps.append(pltpu.async_copy(
                a_hbm.at[idx_s.at[pl.ds(off, _SUB)]],
                ar.at[pl.ds(j * _SUB, _SUB)], gsem))
            cps.append(pltpu.async_copy(
                b_hbm.at[idx_d.at[pl.ds(off, _SUB)]],
                br.at[pl.ds(j * _SUB, _SUB)], gsem))
        return cps

    def compute(ch):
        ar, br, sc = bufs[ch % 2]

        def group(g, gcarry):
            rows = g * 16 + iot
            acc = jnp.zeros((16,), jnp.float32)
            for cc in range(_K):
                av = plsc.load_gather(ar, [rows, kd[cc]])
                bv = plsc.load_gather(br, [rows, kd[cc]])
                acc = acc + jnp.maximum(av + bv, 0.0) * w3d[cc]
            sc[pl.ds(g * 16, 16)] = acc
            return gcarry

        lax.fori_loop(0, _G, group, 0)
        return pltpu.async_copy(sc, out_hbm.at[pl.ds(base + ch * _C, _C)],
                                wsem)

    pend = issue(0)
    writes = []
    for ch in range(_NCH):
        nxt = issue(ch + 1) if ch + 1 < _NCH else []
        for cp in pend:
            cp.wait()
        if ch >= 2:
            writes[ch - 2].wait()
        writes.append(compute(ch))
        pend = nxt
    writes[-2].wait()
    writes[-1].wait()


_edge_call = functools.partial(
    pl.kernel,
    out_type=jax.ShapeDtypeStruct((_E,), jnp.float32),
    mesh=plsc.VectorSubcoreMesh(core_axis_name="c", subcore_axis_name="s",
                                num_cores=_NC, num_subcores=_NS),
    compiler_params=pltpu.CompilerParams(
        needs_layout_passes=False, use_tc_tiling_on_sc=False),
    scratch_types=[
        pltpu.VMEM((_EW,), jnp.int32),
        pltpu.VMEM((_EW,), jnp.int32),
        pltpu.VMEM((_C, _K), jnp.float32),
        pltpu.VMEM((_C, _K), jnp.float32),
        pltpu.VMEM((_C, _K), jnp.float32),
        pltpu.VMEM((_C, _K), jnp.float32),
        pltpu.VMEM((_C,), jnp.float32),
        pltpu.VMEM((_C,), jnp.float32),
        pltpu.VMEM((_K, 16), jnp.float32),
        pltpu.SemaphoreType.DMA,
        pltpu.SemaphoreType.DMA,
    ],
)(_edge_body)


def _norm_body(s_ref, o_ref):
    sv = s_ref[...]
    mn = jnp.min(sv)
    mx = jnp.max(sv)
    o_ref[...] = (sv - mn) / (mx - mn)


_norm_call = pl.pallas_call(
    _norm_body,
    out_shape=jax.ShapeDtypeStruct((_E // 128, 128), jnp.float32),
)


def kernel(h, edge_index, W1, b1, W2, b2, W3, b3):
    a_tab, b_tab, w3b = _tables_call(h, W1, b1, W2, b2, W3)
    src = edge_index[0]
    dst = edge_index[1]
    scores = _edge_call(a_tab, b_tab, src, dst, w3b)
    out2d = _norm_call(scores.reshape(_E // 128, 128))
    return out2d.reshape(_E, 1)


# speedup vs baseline: 21.4635x; 1.1096x over previous
import functools

import jax
import jax.numpy as jnp
from jax import lax
from jax.experimental import pallas as pl
from jax.experimental.pallas import tpu as pltpu
from jax.experimental.pallas import tpu_sc as plsc

_N = 10000
_E = 320000
_D = 128
_H = 16
_K = 8

_NC = 2
_NS = 16
_NW = _NC * _NS
_EW = _E // _NW
_C = 2000
_NCH = _EW // _C
_SUB = 400
_NSUB = _C // _SUB
_G = _C // 16


def _tables_body(h_ref, w1_ref, b1_ref, w2_ref, b2_ref, w3_ref,
                 a_ref, b_ref, w3b_ref):
    h1 = lax.dot_general(h_ref[...], w1_ref[...],
                         (((1,), (1,)), ((), ())),
                         preferred_element_type=jnp.float32) + b1_ref[...][None, :]
    w2 = w2_ref[...]
    a_ref[...] = lax.dot_general(h1, w2[:, :_H],
                                 (((1,), (1,)), ((), ())),
                                 preferred_element_type=jnp.float32) + b2_ref[...][None, :]
    b_ref[...] = lax.dot_general(h1, w2[:, _H:],
                                 (((1,), (1,)), ((), ())),
                                 preferred_element_type=jnp.float32)
    w3b_ref[...] = lax.dot_general(w3_ref[...], jnp.ones((1, 16), jnp.float32),
                                   (((0,), (0,)), ((), ())),
                                   preferred_element_type=jnp.float32)


_tables_call = pl.pallas_call(
    _tables_body,
    out_shape=[
        jax.ShapeDtypeStruct((_N, _K), jnp.float32),
        jax.ShapeDtypeStruct((_N, _K), jnp.float32),
        jax.ShapeDtypeStruct((_K, 16), jnp.float32),
    ],
)


def _edge_body(a_hbm, b_hbm, ei_hbm, w3_hbm, out_hbm,
               idx_s, idx_d, arows0, brows0, arows1, brows1,
               sco0, sco1, w3v, gsem, wsem):
    c = lax.axis_index("c")
    s = lax.axis_index("s")
    wid = s * _NC + c
    base = pl.multiple_of(wid * _EW, 256)

    pltpu.sync_copy(w3_hbm, w3v)
    iot = lax.iota(jnp.int32, 16)
    kd = [(iot + cc) & (_K - 1) for cc in range(_K)]
    w3d = [plsc.load_gather(w3v, [kd[cc], iot]) for cc in range(_K)]

    pltpu.sync_copy(ei_hbm.at[0, pl.ds(base, _EW)], idx_s)
    pltpu.sync_copy(ei_hbm.at[1, pl.ds(base, _EW)], idx_d)

    bufs = [(arows0, brows0, sco0), (arows1, brows1, sco1)]

    def issue(ch):
        ar, br, _ = bufs[ch % 2]
        cps = []
        for j in range(_NSUB):
            off = ch * _C + j * _SUB
            cps.append(pltpu.async_copy(
                a_hbm.at[idx_s.at[pl.ds(off, _SUB)]],
                ar.at[pl.ds(j * _SUB, _SUB)], gsem))
            cps.append(pltpu.async_copy(
                b_hbm.at[idx_d.at[pl.ds(off, _SUB)]],
                br.at[pl.ds(j * _SUB, _SUB)], gsem))
        return cps

    def compute(ch):
        ar, br, sc = bufs[ch % 2]

        def group(g, gcarry):
            rows = g * 16 + iot
            acc = jnp.zeros((16,), jnp.float32)
            for cc in range(_K):
                av = plsc.load_gather(ar, [rows, kd[cc]])
                bv = plsc.load_gather(br, [rows, kd[cc]])
                acc = acc + jnp.maximum(av + bv, 0.0) * w3d[cc]
            sc[pl.ds(g * 16, 16)] = acc
            return gcarry

        lax.fori_loop(0, _G, group, 0)
        return pltpu.async_copy(sc, out_hbm.at[pl.ds(base + ch * _C, _C)],
                                wsem)

    pend = issue(0)
    writes = []
    for ch in range(_NCH):
        nxt = issue(ch + 1) if ch + 1 < _NCH else []
        for cp in pend:
            cp.wait()
        if ch >= 2:
            writes[ch - 2].wait()
        writes.append(compute(ch))
        pend = nxt
    writes[-2].wait()
    writes[-1].wait()


_edge_call = functools.partial(
    pl.kernel,
    out_type=jax.ShapeDtypeStruct((_E,), jnp.float32),
    mesh=plsc.VectorSubcoreMesh(core_axis_name="c", subcore_axis_name="s",
                                num_cores=_NC, num_subcores=_NS),
    compiler_params=pltpu.CompilerParams(
        needs_layout_passes=False, use_tc_tiling_on_sc=False),
    scratch_types=[
        pltpu.VMEM((_EW,), jnp.int32),
        pltpu.VMEM((_EW,), jnp.int32),
        pltpu.VMEM((_C, _K), jnp.float32),
        pltpu.VMEM((_C, _K), jnp.float32),
        pltpu.VMEM((_C, _K), jnp.float32),
        pltpu.VMEM((_C, _K), jnp.float32),
        pltpu.VMEM((_C,), jnp.float32),
        pltpu.VMEM((_C,), jnp.float32),
        pltpu.VMEM((_K, 16), jnp.float32),
        pltpu.SemaphoreType.DMA,
        pltpu.SemaphoreType.DMA,
    ],
)(_edge_body)


def _norm_body(s_ref, o_ref):
    sv = s_ref[...]
    mn = jnp.min(sv)
    mx = jnp.max(sv)
    o_ref[...] = (sv - mn) / (mx - mn)


_norm_call = pl.pallas_call(
    _norm_body,
    out_shape=jax.ShapeDtypeStruct((_E,), jnp.float32),
)


def kernel(h, edge_index, W1, b1, W2, b2, W3, b3):
    a_tab, b_tab, w3b = _tables_call(h, W1, b1, W2, b2, W3)
    scores = _edge_call(a_tab, b_tab, edge_index, w3b)
    return _norm_call(scores).reshape(_E, 1)


# speedup vs baseline: 21.6279x; 1.0077x over previous
import functools

import jax
import jax.numpy as jnp
from jax import lax
from jax.experimental import pallas as pl
from jax.experimental.pallas import tpu as pltpu
from jax.experimental.pallas import tpu_sc as plsc

_N = 10000
_E = 320000
_D = 128
_H = 16
_K = 8

_NC = 2
_NS = 16
_NW = _NC * _NS
_EW = _E // _NW
_C = 2000
_NCH = _EW // _C
_SUB = 400
_NSUB = _C // _SUB
_G = _C // 16


def _tables_body(h_ref, w1_ref, b1_ref, w2_ref, b2_ref, w3_ref,
                 a_ref, b_ref, w3b_ref):
    h1 = lax.dot_general(h_ref[...], w1_ref[...],
                         (((1,), (1,)), ((), ())),
                         preferred_element_type=jnp.float32) + b1_ref[...][None, :]
    w2 = w2_ref[...]
    a_ref[...] = lax.dot_general(h1, w2[:, :_H],
                                 (((1,), (1,)), ((), ())),
                                 preferred_element_type=jnp.float32) + b2_ref[...][None, :]
    b_ref[...] = lax.dot_general(h1, w2[:, _H:],
                                 (((1,), (1,)), ((), ())),
                                 preferred_element_type=jnp.float32)
    w3b_ref[...] = lax.dot_general(w3_ref[...], jnp.ones((1, 16), jnp.float32),
                                   (((0,), (0,)), ((), ())),
                                   preferred_element_type=jnp.float32)


_tables_call = pl.pallas_call(
    _tables_body,
    out_shape=[
        jax.ShapeDtypeStruct((_N, _K), jnp.float32),
        jax.ShapeDtypeStruct((_N, _K), jnp.float32),
        jax.ShapeDtypeStruct((_K, 16), jnp.float32),
    ],
)


def _edge_body(a_hbm, b_hbm, ei_hbm, w3_hbm, out_hbm,
               a_res, idx_s0, idx_s1, idx_d0, idx_d1,
               brows0, brows1, sco0, sco1, w3v,
               asem, isem, gsem, wsem):
    c = lax.axis_index("c")
    s = lax.axis_index("s")
    wid = s * _NC + c
    base = pl.multiple_of(wid * _EW, 256)

    pltpu.sync_copy(w3_hbm, w3v)
    iot = lax.iota(jnp.int32, 16)
    kd = [(iot + cc) & (_K - 1) for cc in range(_K)]
    w3d = [plsc.load_gather(w3v, [kd[cc], iot]) for cc in range(_K)]

    astage = pltpu.async_copy(a_hbm, a_res, asem)

    sbufs = [idx_s0, idx_s1]
    dbufs = [idx_d0, idx_d1]
    bbufs = [brows0, brows1]
    obufs = [sco0, sco1]

    def issue_idx(ch):
        b = ch % 2
        return (pltpu.async_copy(ei_hbm.at[0, pl.ds(base + ch * _C, _C)],
                                 sbufs[b], isem),
                pltpu.async_copy(ei_hbm.at[1, pl.ds(base + ch * _C, _C)],
                                 dbufs[b], isem))

    def issue_gathers(ch):
        b = ch % 2
        cps = []
        for j in range(_NSUB):
            cps.append(pltpu.async_copy(
                b_hbm.at[dbufs[b].at[pl.ds(j * _SUB, _SUB)]],
                bbufs[b].at[pl.ds(j * _SUB, _SUB)], gsem))
        return cps

    def compute(ch):
        b = ch % 2
        sb, br, sc = sbufs[b], bbufs[b], obufs[b]

        def group(g, gcarry):
            sv = sb[pl.ds(pl.multiple_of(g * 16, 16), 16)]
            rows = g * 16 + iot
            terms = []
            for cc in range(_K):
                av = plsc.load_gather(a_res, [sv, kd[cc]])
                bv = plsc.load_gather(br, [rows, kd[cc]])
                terms.append(jnp.maximum(av + bv, 0.0) * w3d[cc])
            t01 = terms[0] + terms[1]
            t23 = terms[2] + terms[3]
            t45 = terms[4] + terms[5]
            t67 = terms[6] + terms[7]
            sc[pl.ds(pl.multiple_of(g * 16, 16), 16)] = (t01 + t23) + (t45 + t67)
            return gcarry

        lax.fori_loop(0, _G, group, 0)
        return pltpu.async_copy(sc, out_hbm.at[pl.ds(base + ch * _C, _C)],
                                wsem)

    idxcp = {0: issue_idx(0)}
    for cp in idxcp[0]:
        cp.wait()
    gath = {0: issue_gathers(0)}
    idxcp[1] = issue_idx(1)
    astage.wait()

    writes = []
    for ch in range(_NCH):
        if ch + 1 < _NCH:
            for cp in idxcp[ch + 1]:
                cp.wait()
            gath[ch + 1] = issue_gathers(ch + 1)
        for cp in gath[ch]:
            cp.wait()
        if ch >= 2:
            writes[ch - 2].wait()
        writes.append(compute(ch))
        if ch + 2 < _NCH:
            idxcp[ch + 2] = issue_idx(ch + 2)
    writes[-2].wait()
    writes[-1].wait()


_edge_call = functools.partial(
    pl.kernel,
    out_type=jax.ShapeDtypeStruct((_E,), jnp.float32),
    mesh=plsc.VectorSubcoreMesh(core_axis_name="c", subcore_axis_name="s",
                                num_cores=_NC, num_subcores=_NS),
    compiler_params=pltpu.CompilerParams(
        needs_layout_passes=False, use_tc_tiling_on_sc=False),
    scratch_types=[
        pltpu.VMEM((_N, _K), jnp.float32),
        pltpu.VMEM((_C,), jnp.int32),
        pltpu.VMEM((_C,), jnp.int32),
        pltpu.VMEM((_C,), jnp.int32),
        pltpu.VMEM((_C,), jnp.int32),
        pltpu.VMEM((_C, _K), jnp.float32),
        pltpu.VMEM((_C, _K), jnp.float32),
        pltpu.VMEM((_C,), jnp.float32),
        pltpu.VMEM((_C,), jnp.float32),
        pltpu.VMEM((_K, 16), jnp.float32),
        pltpu.SemaphoreType.DMA,
        pltpu.SemaphoreType.DMA,
        pltpu.SemaphoreType.DMA,
        pltpu.SemaphoreType.DMA,
    ],
)(_edge_body)


def _norm_body(s_ref, o_ref):
    sv = s_ref[...]
    mn = jnp.min(sv)
    mx = jnp.max(sv)
    o_ref[...] = (sv - mn) / (mx - mn)


_norm_call = pl.pallas_call(
    _norm_body,
    out_shape=jax.ShapeDtypeStruct((_E,), jnp.float32),
)


def kernel(h, edge_index, W1, b1, W2, b2, W3, b3):
    a_tab, b_tab, w3b = _tables_call(h, W1, b1, W2, b2, W3)
    scores = _edge_call(a_tab, b_tab, edge_index, w3b)
    return _norm_call(scores).reshape(_E, 1)
